# Initial kernel scaffold; baseline (speedup 1.0000x reference)
import jax
import jax.numpy as jnp
from jax.experimental import pallas as pl


def kernel(x, edge_index, batch, W0, b0, W1, b1, W2, b2, a0, a1, lin_W, lin_b):
    raise NotImplementedError("write your pallas kernel here")


# speedup vs baseline: 19.1434x; 19.1434x over previous
import functools

import jax
import jax.numpy as jnp
from jax import lax
from jax.experimental import pallas as pl
from jax.experimental.pallas import tpu as pltpu
from jax.experimental.pallas import tpu_sc as plsc

_N, _D, _E, _G, _OUT = 10000, 128, 320000, 64, 10
_D2 = _D // 2
_NC, _NS = 2, 16
_NW = _NC * _NS
_EW = _E // _NW
_K = 100
_ST = _EW // _K
_NP = 10240
_RPT = _NP // _NS

_mesh = plsc.VectorSubcoreMesh(core_axis_name="c", subcore_axis_name="s")


def _sc_spmm_body(tlo_hbm, thi_hbm, src_hbm, dst_hbm, zrow_hbm, out_hbm,
             src_v, dst_v, rows_v, acc_sh, sem0, sem1):
    c = lax.axis_index("c")
    s = lax.axis_index("s")
    wid = c * _NS + s
    pltpu.sync_copy(src_hbm.at[wid], src_v)
    pltpu.sync_copy(dst_hbm.at[wid], dst_v)
    sems = (sem0, sem1)

    for half, t_hbm in ((0, tlo_hbm), (1, thi_hbm)):
        pltpu.sync_copy(zrow_hbm, acc_sh.at[pl.ds(s * _RPT, _RPT)])
        plsc.subcore_barrier()

        for b in range(2):
            pltpu.async_copy(t_hbm.at[src_v.at[b]], rows_v.at[b], sems[b])

        def _body(i, carry):
            for b in range(2):
                j = 2 * i + b
                pltpu.make_async_copy(
                    t_hbm.at[src_v.at[j]], rows_v.at[b], sems[b]).wait()
                pltpu.sync_copy(rows_v.at[b], acc_sh.at[dst_v.at[j]],
                                add=True)
                pltpu.async_copy(
                    t_hbm.at[src_v.at[j + 2]], rows_v.at[b], sems[b])
            return carry

        lax.fori_loop(0, _ST // 2 - 1, _body, 0)
        for b in range(2):
            j = _ST - 2 + b
            pltpu.make_async_copy(
                t_hbm.at[src_v.at[j]], rows_v.at[b], sems[b]).wait()
            pltpu.sync_copy(rows_v.at[b], acc_sh.at[dst_v.at[j]], add=True)

        plsc.subcore_barrier()
        pltpu.sync_copy(acc_sh.at[pl.ds(s * _RPT, _RPT)],
                        out_hbm.at[c, half, pl.ds(s * _RPT, _RPT)])


def _make_sc_spmm(interpret=False):
    return functools.partial(
        pl.kernel,
        out_type=jax.ShapeDtypeStruct((_NC, 2, _NP, _D2), jnp.float32),
        mesh=_mesh,
        interpret=interpret,
        compiler_params=pltpu.CompilerParams(use_tc_tiling_on_sc=False),
        scratch_types=[
            pltpu.VMEM((_ST, _K), jnp.int32),
            pltpu.VMEM((_ST, _K), jnp.int32),
            pltpu.VMEM((2, _K, _D2), jnp.float32),
            pltpu.VMEM_SHARED((_NP, _D2), jnp.float32),
            pltpu.SemaphoreType.DMA,
            pltpu.SemaphoreType.DMA,
        ],
    )(_sc_spmm_body)


_sc_spmm = _make_sc_spmm()


def _sc_deg_body(dst_hbm, zrow_hbm, ones_hbm, out_hbm, dst_v, ones_v, acc_sh):
    c = lax.axis_index("c")
    s = lax.axis_index("s")
    wid = c * _NS + s
    pltpu.sync_copy(zrow_hbm, acc_sh.at[pl.ds(s * _RPT, _RPT)])
    pltpu.sync_copy(dst_hbm.at[wid], dst_v)
    pltpu.sync_copy(ones_hbm, ones_v)
    plsc.subcore_barrier()

    def _body(j, carry):
        pltpu.sync_copy(ones_v, acc_sh.at[dst_v.at[j]], add=True)
        return carry

    lax.fori_loop(0, _ST, _body, 0)
    plsc.subcore_barrier()
    pltpu.sync_copy(acc_sh.at[pl.ds(s * _RPT, _RPT)],
                    out_hbm.at[c, pl.ds(s * _RPT, _RPT)])


def _make_sc_deg(interpret=False):
    return functools.partial(
        pl.kernel,
        out_type=jax.ShapeDtypeStruct((_NC, _NP, 16), jnp.float32),
        mesh=_mesh,
        interpret=interpret,
        compiler_params=pltpu.CompilerParams(use_tc_tiling_on_sc=False),
        scratch_types=[
            pltpu.VMEM((_ST, _K), jnp.int32),
            pltpu.VMEM((_K, 16), jnp.float32),
            pltpu.VMEM_SHARED((_NP, 16), jnp.float32),
        ],
    )(_sc_deg_body)


_sc_deg = _make_sc_deg()


_R = 1000
_P_SPECS = [
    pl.BlockSpec((1, 1, _R, _D2), lambda i, c=c, h=h: (c, h, i, 0))
    for c in range(_NC) for h in range(2)
]


def _psum(p_refs):
    lo = p_refs[0][0, 0] + p_refs[2][0, 0]
    hi = p_refs[1][0, 0] + p_refs[3][0, 0]
    return lo, hi


def _k0_body(dp0_ref, dp1_ref, x_ref, dinv_ref, tlo_ref, thi_ref):
    deg = dp0_ref[:, :1] + dp1_ref[:, :1] + 1.0
    dinv = lax.rsqrt(deg)
    dinv_ref[...] = dinv
    tlo_ref[...] = x_ref[:, :_D2] * dinv
    thi_ref[...] = x_ref[:, _D2:] * dinv


def _tc_prescale(dp0, dp1, x):
    return pl.pallas_call(
        _k0_body,
        grid=(_N // _R,),
        in_specs=[
            pl.BlockSpec((_R, 16), lambda i: (i, 0)),
            pl.BlockSpec((_R, 16), lambda i: (i, 0)),
            pl.BlockSpec((_R, _D), lambda i: (i, 0)),
        ],
        out_specs=[
            pl.BlockSpec((_R, 1), lambda i: (i, 0)),
            pl.BlockSpec((_R, _D2), lambda i: (i, 0)),
            pl.BlockSpec((_R, _D2), lambda i: (i, 0)),
        ],
        out_shape=[
            jax.ShapeDtypeStruct((_N, 1), jnp.float32),
            jax.ShapeDtypeStruct((_N, _D2), jnp.float32),
            jax.ShapeDtypeStruct((_N, _D2), jnp.float32),
        ],
    )(dp0, dp1, x)


def _klayer_body(p00_ref, p01_ref, p10_ref, p11_ref, tlo_ref, thi_ref,
                 dinv_ref, w_ref, b_ref, a_ref, olo_ref, ohi_ref):
    dinv = dinv_ref[...]
    plo, phi = _psum((p00_ref, p01_ref, p10_ref, p11_ref))
    slo = dinv * (plo + tlo_ref[...])
    shi = dinv * (phi + thi_ref[...])
    h = (jnp.dot(slo, w_ref[:_D2, :], preferred_element_type=jnp.float32)
         + jnp.dot(shi, w_ref[_D2:, :], preferred_element_type=jnp.float32)
         + b_ref[...])
    a = a_ref[0, 0]
    act = jnp.where(h > 0, h, a * h)
    out = act * dinv
    olo_ref[...] = out[:, :_D2]
    ohi_ref[...] = out[:, _D2:]


def _tc_layer(p, tlo, thi, dinv, w, b, a):
    return pl.pallas_call(
        _klayer_body,
        grid=(_N // _R,),
        in_specs=_P_SPECS + [
            pl.BlockSpec((_R, _D2), lambda i: (i, 0)),
            pl.BlockSpec((_R, _D2), lambda i: (i, 0)),
            pl.BlockSpec((_R, 1), lambda i: (i, 0)),
            pl.BlockSpec((_D, _D), lambda i: (0, 0)),
            pl.BlockSpec((1, _D), lambda i: (0, 0)),
            pl.BlockSpec((1, 1), lambda i: (0, 0)),
        ],
        out_specs=[
            pl.BlockSpec((_R, _D2), lambda i: (i, 0)),
            pl.BlockSpec((_R, _D2), lambda i: (i, 0)),
        ],
        out_shape=[
            jax.ShapeDtypeStruct((_N, _D2), jnp.float32),
            jax.ShapeDtypeStruct((_N, _D2), jnp.float32),
        ],
    )(p, p, p, p, tlo, thi, dinv, w, b, a)


def _kfinal_body(p00_ref, p01_ref, p10_ref, p11_ref, tlo_ref, thi_ref,
                 dinv_ref, w_ref, b_ref, batch_ref, lw_ref, lb_ref,
                 out_ref, pooled_acc, cnt_acc):
    i = pl.program_id(0)

    @pl.when(i == 0)
    def _init():
        pooled_acc[...] = jnp.zeros_like(pooled_acc)
        cnt_acc[...] = jnp.zeros_like(cnt_acc)

    dinv = dinv_ref[...]
    plo, phi = _psum((p00_ref, p01_ref, p10_ref, p11_ref))
    slo = dinv * (plo + tlo_ref[...])
    shi = dinv * (phi + thi_ref[...])
    h = (jnp.dot(slo, w_ref[:_D2, :], preferred_element_type=jnp.float32)
         + jnp.dot(shi, w_ref[_D2:, :], preferred_element_type=jnp.float32)
         + b_ref[...])
    m = (batch_ref[...] ==
         lax.broadcasted_iota(jnp.int32, (_R, _G), 1)).astype(jnp.float32)
    dn = (((0,), (0,)), ((), ()))
    pooled_acc[...] += lax.dot_general(
        m, h, dn, preferred_element_type=jnp.float32)
    cnt_acc[...] += lax.dot_general(
        m, jnp.ones((_R, 1), jnp.float32), dn,
        preferred_element_type=jnp.float32)

    @pl.when(i == pl.num_programs(0) - 1)
    def _fin():
        pooled = pooled_acc[...] / jnp.maximum(cnt_acc[...], 1.0)
        logits = jnp.dot(pooled, lw_ref[...],
                         preferred_element_type=jnp.float32) + lb_ref[...]
        mx = jnp.max(logits, axis=1, keepdims=True)
        lse = jnp.log(jnp.sum(jnp.exp(logits - mx), axis=1,
                              keepdims=True)) + mx
        out_ref[...] = logits - lse


def _tc_final(p, tlo, thi, dinv, w, b, batch2d, lw, lb):
    return pl.pallas_call(
        _kfinal_body,
        grid=(_N // _R,),
        in_specs=_P_SPECS + [
            pl.BlockSpec((_R, _D2), lambda i: (i, 0)),
            pl.BlockSpec((_R, _D2), lambda i: (i, 0)),
            pl.BlockSpec((_R, 1), lambda i: (i, 0)),
            pl.BlockSpec((_D, _D), lambda i: (0, 0)),
            pl.BlockSpec((1, _D), lambda i: (0, 0)),
            pl.BlockSpec((_R, 1), lambda i: (i, 0)),
            pl.BlockSpec((_D, _OUT), lambda i: (0, 0)),
            pl.BlockSpec((1, _OUT), lambda i: (0, 0)),
        ],
        out_specs=pl.BlockSpec((_G, _OUT), lambda i: (0, 0)),
        out_shape=jax.ShapeDtypeStruct((_G, _OUT), jnp.float32),
        scratch_shapes=[
            pltpu.VMEM((_G, _D), jnp.float32),
            pltpu.VMEM((_G, 1), jnp.float32),
        ],
    )(p, p, p, p, tlo, thi, dinv, w, b, batch2d, lw, lb)


def kernel(x, edge_index, batch, W0, b0, W1, b1, W2, b2, a0, a1, lin_W,
           lin_b):
    src = edge_index[0].reshape(_NW, _ST, _K)
    dst = edge_index[1].reshape(_NW, _ST, _K)
    zrow_d = jnp.zeros((_RPT, _D2), jnp.float32)
    zrow_16 = jnp.zeros((_RPT, 16), jnp.float32)
    ones_16 = jnp.ones((_K, 16), jnp.float32)
    batch2d = batch.reshape(_N, 1)
    b0r, b1r, b2r = (v.reshape(1, _D) for v in (b0, b1, b2))
    a0r, a1r = a0.reshape(1, 1), a1.reshape(1, 1)
    lbr = lin_b.reshape(1, _OUT)

    degp = _sc_deg(dst, zrow_16, ones_16)
    dinv, tlo, thi = _tc_prescale(degp[0], degp[1], x)
    p = _sc_spmm(tlo, thi, src, dst, zrow_d)
    tlo, thi = _tc_layer(p, tlo, thi, dinv, W0, b0r, a0r)
    p = _sc_spmm(tlo, thi, src, dst, zrow_d)
    tlo, thi = _tc_layer(p, tlo, thi, dinv, W1, b1r, a1r)
    p = _sc_spmm(tlo, thi, src, dst, zrow_d)
    return _tc_final(p, tlo, thi, dinv, W2, b2r, batch2d, lin_W, lbr)


# speedup vs baseline: 21.1639x; 1.1055x over previous
import functools

import jax
import jax.numpy as jnp
from jax import lax
from jax.experimental import pallas as pl
from jax.experimental.pallas import tpu as pltpu
from jax.experimental.pallas import tpu_sc as plsc

_N, _D, _E, _G, _OUT = 10000, 128, 320000, 64, 10
_D2 = _D // 2
_NC, _NS = 2, 16
_NW = _NC * _NS
_EW = _E // _NW
_K = 125
_ST = _EW // _K
_NB = 4
_PF = 2
_NP = 10240
_RPT = _NP // _NS

_mesh = plsc.VectorSubcoreMesh(core_axis_name="c", subcore_axis_name="s")


def _sc_spmm_body(tlo_hbm, thi_hbm, src_hbm, dst_hbm, zrow_hbm, out_hbm,
                  src_v, dst_v, rows_v, acc_sh,
                  gs0, gs1, gs2, gs3, ss0, ss1, ss2, ss3):
    c = lax.axis_index("c")
    s = lax.axis_index("s")
    wid = c * _NS + s
    gs = (gs0, gs1, gs2, gs3)
    ss = (ss0, ss1, ss2, ss3)
    pltpu.sync_copy(src_hbm.at[wid], src_v)
    pltpu.sync_copy(dst_hbm.at[wid], dst_v)

    for half, t_hbm in ((0, tlo_hbm), (1, thi_hbm)):
        pltpu.sync_copy(zrow_hbm, acc_sh.at[pl.ds(s * _RPT, _RPT)])
        plsc.subcore_barrier()

        def _gather(j, b):
            pltpu.async_copy(t_hbm.at[src_v.at[j]], rows_v.at[b], gs[b])

        def _gwait(j, b):
            pltpu.make_async_copy(
                t_hbm.at[src_v.at[j]], rows_v.at[b], gs[b]).wait()

        def _scat(j, b):
            pltpu.async_copy(rows_v.at[b], acc_sh.at[dst_v.at[j]], ss[b],
                             add=True)

        def _swait(j, b):
            pltpu.make_async_copy(
                rows_v.at[b], acc_sh.at[dst_v.at[j]], ss[b]).wait()

        for j in range(_PF):
            _gather(j, j % _NB)
        for b in range(_NB):
            _gwait(b, b)
            _scat(b, b)
            k = b + _PF
            if k >= _NB:
                _swait(k - _NB, k % _NB)
            _gather(k, k % _NB)

        def _body(i, carry):
            for b in range(_NB):
                j = _NB * i + b
                kb = (b + _PF) % _NB
                _gwait(j, b)
                _scat(j, b)
                _swait(j + _PF - _NB, kb)
                _gather(j + _PF, kb)
            return carry

        lax.fori_loop(1, _ST // _NB - 1, _body, 0)
        for b in range(_NB):
            j = _ST - _NB + b
            _gwait(j, b)
            _scat(j, b)
            k = j + _PF
            if k < _ST:
                _swait(k - _NB, k % _NB)
                _gather(k, k % _NB)
        for b in range(_NB):
            _swait(_ST - _NB + b, b)

        plsc.subcore_barrier()
        pltpu.sync_copy(acc_sh.at[pl.ds(s * _RPT, _RPT)],
                        out_hbm.at[c, half, pl.ds(s * _RPT, _RPT)])


def _make_sc_spmm(interpret=False):
    return functools.partial(
        pl.kernel,
        out_type=jax.ShapeDtypeStruct((_NC, 2, _NP, _D2), jnp.float32),
        mesh=_mesh,
        interpret=interpret,
        compiler_params=pltpu.CompilerParams(use_tc_tiling_on_sc=False),
        scratch_types=[
            pltpu.VMEM((_ST, _K), jnp.int32),
            pltpu.VMEM((_ST, _K), jnp.int32),
            pltpu.VMEM((_NB, _K, _D2), jnp.float32),
            pltpu.VMEM_SHARED((_NP, _D2), jnp.float32),
        ] + [pltpu.SemaphoreType.DMA] * (2 * _NB),
    )(_sc_spmm_body)


_sc_spmm = _make_sc_spmm()


def _sc_deg_body(dst_hbm, zrow_hbm, ones_hbm, out_hbm, dst_v, ones_v, acc_sh):
    c = lax.axis_index("c")
    s = lax.axis_index("s")
    wid = c * _NS + s
    pltpu.sync_copy(zrow_hbm, acc_sh.at[pl.ds(s * _RPT, _RPT)])
    pltpu.sync_copy(dst_hbm.at[wid], dst_v)
    pltpu.sync_copy(ones_hbm, ones_v)
    plsc.subcore_barrier()

    def _body(j, carry):
        pltpu.sync_copy(ones_v, acc_sh.at[dst_v.at[j]], add=True)
        return carry

    lax.fori_loop(0, _ST, _body, 0)
    plsc.subcore_barrier()
    pltpu.sync_copy(acc_sh.at[pl.ds(s * _RPT, _RPT)],
                    out_hbm.at[c, pl.ds(s * _RPT, _RPT)])


def _make_sc_deg(interpret=False):
    return functools.partial(
        pl.kernel,
        out_type=jax.ShapeDtypeStruct((_NC, _NP, 16), jnp.float32),
        mesh=_mesh,
        interpret=interpret,
        compiler_params=pltpu.CompilerParams(use_tc_tiling_on_sc=False),
        scratch_types=[
            pltpu.VMEM((_ST, _K), jnp.int32),
            pltpu.VMEM((_K, 16), jnp.float32),
            pltpu.VMEM_SHARED((_NP, 16), jnp.float32),
        ],
    )(_sc_deg_body)


_sc_deg = _make_sc_deg()


_R = 1000
_P_SPECS = [
    pl.BlockSpec((1, 1, _R, _D2), lambda i, c=c, h=h: (c, h, i, 0))
    for c in range(_NC) for h in range(2)
]


def _psum(p_refs):
    lo = p_refs[0][0, 0] + p_refs[2][0, 0]
    hi = p_refs[1][0, 0] + p_refs[3][0, 0]
    return lo, hi


def _k0_body(dp0_ref, dp1_ref, x_ref, dinv_ref, tlo_ref, thi_ref):
    deg = dp0_ref[:, :1] + dp1_ref[:, :1] + 1.0
    dinv = lax.rsqrt(deg)
    dinv_ref[...] = dinv
    tlo_ref[...] = x_ref[:, :_D2] * dinv
    thi_ref[...] = x_ref[:, _D2:] * dinv


def _tc_prescale(dp0, dp1, x):
    return pl.pallas_call(
        _k0_body,
        grid=(_N // _R,),
        in_specs=[
            pl.BlockSpec((_R, 16), lambda i: (i, 0)),
            pl.BlockSpec((_R, 16), lambda i: (i, 0)),
            pl.BlockSpec((_R, _D), lambda i: (i, 0)),
        ],
        out_specs=[
            pl.BlockSpec((_R, 1), lambda i: (i, 0)),
            pl.BlockSpec((_R, _D2), lambda i: (i, 0)),
            pl.BlockSpec((_R, _D2), lambda i: (i, 0)),
        ],
        out_shape=[
            jax.ShapeDtypeStruct((_N, 1), jnp.float32),
            jax.ShapeDtypeStruct((_N, _D2), jnp.float32),
            jax.ShapeDtypeStruct((_N, _D2), jnp.float32),
        ],
    )(dp0, dp1, x)


def _klayer_body(p00_ref, p01_ref, p10_ref, p11_ref, tlo_ref, thi_ref,
                 dinv_ref, w_ref, b_ref, a_ref, olo_ref, ohi_ref):
    dinv = dinv_ref[...]
    plo, phi = _psum((p00_ref, p01_ref, p10_ref, p11_ref))
    slo = dinv * (plo + tlo_ref[...])
    shi = dinv * (phi + thi_ref[...])
    h = (jnp.dot(slo, w_ref[:_D2, :], preferred_element_type=jnp.float32)
         + jnp.dot(shi, w_ref[_D2:, :], preferred_element_type=jnp.float32)
         + b_ref[...])
    a = a_ref[0, 0]
    act = jnp.where(h > 0, h, a * h)
    out = act * dinv
    olo_ref[...] = out[:, :_D2]
    ohi_ref[...] = out[:, _D2:]


def _tc_layer(p, tlo, thi, dinv, w, b, a):
    return pl.pallas_call(
        _klayer_body,
        grid=(_N // _R,),
        in_specs=_P_SPECS + [
            pl.BlockSpec((_R, _D2), lambda i: (i, 0)),
            pl.BlockSpec((_R, _D2), lambda i: (i, 0)),
            pl.BlockSpec((_R, 1), lambda i: (i, 0)),
            pl.BlockSpec((_D, _D), lambda i: (0, 0)),
            pl.BlockSpec((1, _D), lambda i: (0, 0)),
            pl.BlockSpec((1, 1), lambda i: (0, 0)),
        ],
        out_specs=[
            pl.BlockSpec((_R, _D2), lambda i: (i, 0)),
            pl.BlockSpec((_R, _D2), lambda i: (i, 0)),
        ],
        out_shape=[
            jax.ShapeDtypeStruct((_N, _D2), jnp.float32),
            jax.ShapeDtypeStruct((_N, _D2), jnp.float32),
        ],
    )(p, p, p, p, tlo, thi, dinv, w, b, a)


def _kfinal_body(p00_ref, p01_ref, p10_ref, p11_ref, tlo_ref, thi_ref,
                 dinv_ref, w_ref, b_ref, batch_ref, lw_ref, lb_ref,
                 out_ref, pooled_acc, cnt_acc):
    i = pl.program_id(0)

    @pl.when(i == 0)
    def _init():
        pooled_acc[...] = jnp.zeros_like(pooled_acc)
        cnt_acc[...] = jnp.zeros_like(cnt_acc)

    dinv = dinv_ref[...]
    plo, phi = _psum((p00_ref, p01_ref, p10_ref, p11_ref))
    slo = dinv * (plo + tlo_ref[...])
    shi = dinv * (phi + thi_ref[...])
    h = (jnp.dot(slo, w_ref[:_D2, :], preferred_element_type=jnp.float32)
         + jnp.dot(shi, w_ref[_D2:, :], preferred_element_type=jnp.float32)
         + b_ref[...])
    m = (batch_ref[...] ==
         lax.broadcasted_iota(jnp.int32, (_R, _G), 1)).astype(jnp.float32)
    dn = (((0,), (0,)), ((), ()))
    pooled_acc[...] += lax.dot_general(
        m, h, dn, preferred_element_type=jnp.float32)
    cnt_acc[...] += lax.dot_general(
        m, jnp.ones((_R, 1), jnp.float32), dn,
        preferred_element_type=jnp.float32)

    @pl.when(i == pl.num_programs(0) - 1)
    def _fin():
        pooled = pooled_acc[...] / jnp.maximum(cnt_acc[...], 1.0)
        logits = jnp.dot(pooled, lw_ref[...],
                         preferred_element_type=jnp.float32) + lb_ref[...]
        mx = jnp.max(logits, axis=1, keepdims=True)
        lse = jnp.log(jnp.sum(jnp.exp(logits - mx), axis=1,
                              keepdims=True)) + mx
        out_ref[...] = logits - lse


def _tc_final(p, tlo, thi, dinv, w, b, batch2d, lw, lb):
    return pl.pallas_call(
        _kfinal_body,
        grid=(_N // _R,),
        in_specs=_P_SPECS + [
            pl.BlockSpec((_R, _D2), lambda i: (i, 0)),
            pl.BlockSpec((_R, _D2), lambda i: (i, 0)),
            pl.BlockSpec((_R, 1), lambda i: (i, 0)),
            pl.BlockSpec((_D, _D), lambda i: (0, 0)),
            pl.BlockSpec((1, _D), lambda i: (0, 0)),
            pl.BlockSpec((_R, 1), lambda i: (i, 0)),
            pl.BlockSpec((_D, _OUT), lambda i: (0, 0)),
            pl.BlockSpec((1, _OUT), lambda i: (0, 0)),
        ],
        out_specs=pl.BlockSpec((_G, _OUT), lambda i: (0, 0)),
        out_shape=jax.ShapeDtypeStruct((_G, _OUT), jnp.float32),
        scratch_shapes=[
            pltpu.VMEM((_G, _D), jnp.float32),
            pltpu.VMEM((_G, 1), jnp.float32),
        ],
    )(p, p, p, p, tlo, thi, dinv, w, b, batch2d, lw, lb)


def kernel(x, edge_index, batch, W0, b0, W1, b1, W2, b2, a0, a1, lin_W,
           lin_b):
    src = edge_index[0].reshape(_NW, _ST, _K)
    dst = edge_index[1].reshape(_NW, _ST, _K)
    zrow_d = jnp.zeros((_RPT, _D2), jnp.float32)
    zrow_16 = jnp.zeros((_RPT, 16), jnp.float32)
    ones_16 = jnp.ones((_K, 16), jnp.float32)
    batch2d = batch.reshape(_N, 1)
    b0r, b1r, b2r = (v.reshape(1, _D) for v in (b0, b1, b2))
    a0r, a1r = a0.reshape(1, 1), a1.reshape(1, 1)
    lbr = lin_b.reshape(1, _OUT)

    degp = _sc_deg(dst, zrow_16, ones_16)
    dinv, tlo, thi = _tc_prescale(degp[0], degp[1], x)
    p = _sc_spmm(tlo, thi, src, dst, zrow_d)
    tlo, thi = _tc_layer(p, tlo, thi, dinv, W0, b0r, a0r)
    p = _sc_spmm(tlo, thi, src, dst, zrow_d)
    tlo, thi = _tc_layer(p, tlo, thi, dinv, W1, b1r, a1r)
    p = _sc_spmm(tlo, thi, src, dst, zrow_d)
    return _tc_final(p, tlo, thi, dinv, W2, b2r, batch2d, lin_W, lbr)


# speedup vs baseline: 22.2266x; 1.0502x over previous
import functools

import jax
import jax.numpy as jnp
from jax import lax
from jax.experimental import pallas as pl
from jax.experimental.pallas import tpu as pltpu
from jax.experimental.pallas import tpu_sc as plsc

_N, _D, _E, _G, _OUT = 10000, 128, 320000, 64, 10
_D2 = _D // 2
_NC, _NS = 2, 16
_NW = _NC * _NS
_K = 125
_EWS = _E // _NS
_STS = _EWS // _K
_EWD = _E // _NW
_STD = _EWD // _K
_NB = 4
_PF = 2
_NP = 10240
_RPT = _NP // _NS

_mesh = plsc.VectorSubcoreMesh(core_axis_name="c", subcore_axis_name="s")


def _sc_spmm_body(t_hbm, src_hbm, dst_hbm, zrow_hbm, out_hbm,
                  src_v, dst_v, rows_v, acc_sh,
                  gs0, gs1, gs2, gs3, ss0, ss1, ss2, ss3):
    c = lax.axis_index("c")
    s = lax.axis_index("s")
    gs = (gs0, gs1, gs2, gs3)
    ss = (ss0, ss1, ss2, ss3)
    th = t_hbm.at[c]
    pltpu.sync_copy(src_hbm.at[s], src_v)
    pltpu.sync_copy(dst_hbm.at[s], dst_v)
    pltpu.sync_copy(zrow_hbm, acc_sh.at[pl.ds(s * _RPT, _RPT)])
    plsc.subcore_barrier()

    def _gather(j, b):
        pltpu.async_copy(th.at[src_v.at[j]], rows_v.at[b], gs[b])

    def _gwait(j, b):
        pltpu.make_async_copy(th.at[src_v.at[j]], rows_v.at[b], gs[b]).wait()

    def _scat(j, b):
        pltpu.async_copy(rows_v.at[b], acc_sh.at[dst_v.at[j]], ss[b],
                         add=True)
        pltpu.make_async_copy(
            rows_v.at[b], acc_sh.at[dst_v.at[j]], ss[b]).wait()

    for j in range(_PF):
        _gather(j, j % _NB)

    def _body(i, carry):
        for b in range(_NB):
            j = _NB * i + b
            _gwait(j, b)
            _scat(j, b)
            _gather(j + _PF, (b + _PF) % _NB)
        return carry

    lax.fori_loop(0, _STS // _NB - 1, _body, 0)
    for b in range(_NB):
        j = _STS - _NB + b
        _gwait(j, b)
        _scat(j, b)
        k = j + _PF
        if k < _STS:
            _gather(k, k % _NB)

    plsc.subcore_barrier()
    pltpu.sync_copy(acc_sh.at[pl.ds(s * _RPT, _RPT)],
                    out_hbm.at[c, pl.ds(s * _RPT, _RPT)])


def _make_sc_spmm(interpret=False):
    return functools.partial(
        pl.kernel,
        out_type=jax.ShapeDtypeStruct((_NC, _NP, _D2), jnp.float32),
        mesh=_mesh,
        interpret=interpret,
        compiler_params=pltpu.CompilerParams(use_tc_tiling_on_sc=False),
        scratch_types=[
            pltpu.VMEM((_STS, _K), jnp.int32),
            pltpu.VMEM((_STS, _K), jnp.int32),
            pltpu.VMEM((_NB, _K, _D2), jnp.float32),
            pltpu.VMEM_SHARED((_NP, _D2), jnp.float32),
        ] + [pltpu.SemaphoreType.DMA] * (2 * _NB),
    )(_sc_spmm_body)


_sc_spmm = _make_sc_spmm()


def _sc_deg_body(dst_hbm, zrow_hbm, ones_hbm, out_hbm, dst_v, ones_v,
                 acc_sh):
    c = lax.axis_index("c")
    s = lax.axis_index("s")
    wid = c * _NS + s
    pltpu.sync_copy(zrow_hbm, acc_sh.at[pl.ds(s * _RPT, _RPT)])
    pltpu.sync_copy(dst_hbm.at[wid], dst_v)
    pltpu.sync_copy(ones_hbm, ones_v)
    plsc.subcore_barrier()

    def _body(j, carry):
        pltpu.sync_copy(ones_v, acc_sh.at[dst_v.at[j]], add=True)
        return carry

    lax.fori_loop(0, _STD, _body, 0)
    plsc.subcore_barrier()
    pltpu.sync_copy(acc_sh.at[pl.ds(s * _RPT, _RPT)],
                    out_hbm.at[c, pl.ds(s * _RPT, _RPT)])


def _make_sc_deg(interpret=False):
    return functools.partial(
        pl.kernel,
        out_type=jax.ShapeDtypeStruct((_NC, _NP, 16), jnp.float32),
        mesh=_mesh,
        interpret=interpret,
        compiler_params=pltpu.CompilerParams(use_tc_tiling_on_sc=False),
        scratch_types=[
            pltpu.VMEM((_STD, _K), jnp.int32),
            pltpu.VMEM((_K, 16), jnp.float32),
            pltpu.VMEM_SHARED((_NP, 16), jnp.float32),
        ],
    )(_sc_deg_body)


_sc_deg = _make_sc_deg()


_R = 1000
_PT_SPECS = [
    pl.BlockSpec((1, _R, _D2), lambda i, h=h: (h, i, 0)) for h in range(2)
]


def _k0_body(dp0_ref, dp1_ref, x_ref, dinv_ref, t_ref):
    deg = dp0_ref[:, :1] + dp1_ref[:, :1] + 1.0
    dinv = lax.rsqrt(deg)
    dinv_ref[...] = dinv
    t_ref[0] = x_ref[:, :_D2] * dinv
    t_ref[1] = x_ref[:, _D2:] * dinv


def _tc_prescale(dp0, dp1, x):
    return pl.pallas_call(
        _k0_body,
        grid=(_N // _R,),
        in_specs=[
            pl.BlockSpec((_R, 16), lambda i: (i, 0)),
            pl.BlockSpec((_R, 16), lambda i: (i, 0)),
            pl.BlockSpec((_R, _D), lambda i: (i, 0)),
        ],
        out_specs=[
            pl.BlockSpec((_R, 1), lambda i: (i, 0)),
            pl.BlockSpec((2, _R, _D2), lambda i: (0, i, 0)),
        ],
        out_shape=[
            jax.ShapeDtypeStruct((_N, 1), jnp.float32),
            jax.ShapeDtypeStruct((2, _N, _D2), jnp.float32),
        ],
    )(dp0, dp1, x)


def _klayer_body(plo_ref, phi_ref, t_ref, dinv_ref, w_ref, b_ref, a_ref,
                 out_ref):
    dinv = dinv_ref[...]
    slo = dinv * (plo_ref[0] + t_ref[0])
    shi = dinv * (phi_ref[0] + t_ref[1])
    h = (jnp.dot(slo, w_ref[:_D2, :], preferred_element_type=jnp.float32)
         + jnp.dot(shi, w_ref[_D2:, :], preferred_element_type=jnp.float32)
         + b_ref[...])
    a = a_ref[0, 0]
    act = jnp.where(h > 0, h, a * h)
    out = act * dinv
    out_ref[0] = out[:, :_D2]
    out_ref[1] = out[:, _D2:]


def _tc_layer(p, t, dinv, w, b, a):
    return pl.pallas_call(
        _klayer_body,
        grid=(_N // _R,),
        in_specs=_PT_SPECS + [
            pl.BlockSpec((2, _R, _D2), lambda i: (0, i, 0)),
            pl.BlockSpec((_R, 1), lambda i: (i, 0)),
            pl.BlockSpec((_D, _D), lambda i: (0, 0)),
            pl.BlockSpec((1, _D), lambda i: (0, 0)),
            pl.BlockSpec((1, 1), lambda i: (0, 0)),
        ],
        out_specs=pl.BlockSpec((2, _R, _D2), lambda i: (0, i, 0)),
        out_shape=jax.ShapeDtypeStruct((2, _N, _D2), jnp.float32),
    )(p, p, t, dinv, w, b, a)


def _kfinal_body(plo_ref, phi_ref, t_ref, dinv_ref, w_ref, b_ref,
                 batch_ref, lw_ref, lb_ref, out_ref, pooled_acc, cnt_acc):
    i = pl.program_id(0)

    @pl.when(i == 0)
    def _init():
        pooled_acc[...] = jnp.zeros_like(pooled_acc)
        cnt_acc[...] = jnp.zeros_like(cnt_acc)

    dinv = dinv_ref[...]
    slo = dinv * (plo_ref[0] + t_ref[0])
    shi = dinv * (phi_ref[0] + t_ref[1])
    h = (jnp.dot(slo, w_ref[:_D2, :], preferred_element_type=jnp.float32)
         + jnp.dot(shi, w_ref[_D2:, :], preferred_element_type=jnp.float32)
         + b_ref[...])
    m = (batch_ref[...] ==
         lax.broadcasted_iota(jnp.int32, (_R, _G), 1)).astype(jnp.float32)
    dn = (((0,), (0,)), ((), ()))
    pooled_acc[...] += lax.dot_general(
        m, h, dn, preferred_element_type=jnp.float32)
    cnt_acc[...] += lax.dot_general(
        m, jnp.ones((_R, 1), jnp.float32), dn,
        preferred_element_type=jnp.float32)

    @pl.when(i == pl.num_programs(0) - 1)
    def _fin():
        pooled = pooled_acc[...] / jnp.maximum(cnt_acc[...], 1.0)
        logits = jnp.dot(pooled, lw_ref[...],
                         preferred_element_type=jnp.float32) + lb_ref[...]
        mx = jnp.max(logits, axis=1, keepdims=True)
        lse = jnp.log(jnp.sum(jnp.exp(logits - mx), axis=1,
                              keepdims=True)) + mx
        out_ref[...] = logits - lse


def _tc_final(p, t, dinv, w, b, batch2d, lw, lb):
    return pl.pallas_call(
        _kfinal_body,
        grid=(_N // _R,),
        in_specs=_PT_SPECS + [
            pl.BlockSpec((2, _R, _D2), lambda i: (0, i, 0)),
            pl.BlockSpec((_R, 1), lambda i: (i, 0)),
            pl.BlockSpec((_D, _D), lambda i: (0, 0)),
            pl.BlockSpec((1, _D), lambda i: (0, 0)),
            pl.BlockSpec((_R, 1), lambda i: (i, 0)),
            pl.BlockSpec((_D, _OUT), lambda i: (0, 0)),
            pl.BlockSpec((1, _OUT), lambda i: (0, 0)),
        ],
        out_specs=pl.BlockSpec((_G, _OUT), lambda i: (0, 0)),
        out_shape=jax.ShapeDtypeStruct((_G, _OUT), jnp.float32),
        scratch_shapes=[
            pltpu.VMEM((_G, _D), jnp.float32),
            pltpu.VMEM((_G, 1), jnp.float32),
        ],
    )(p, p, t, dinv, w, b, batch2d, lw, lb)


def kernel(x, edge_index, batch, W0, b0, W1, b1, W2, b2, a0, a1, lin_W,
           lin_b):
    src_s = edge_index[0].reshape(_NS, _STS, _K)
    dst_s = edge_index[1].reshape(_NS, _STS, _K)
    dst_d = edge_index[1].reshape(_NW, _STD, _K)
    zrow_d = jnp.zeros((_RPT, _D2), jnp.float32)
    zrow_16 = jnp.zeros((_RPT, 16), jnp.float32)
    ones_16 = jnp.ones((_K, 16), jnp.float32)
    batch2d = batch.reshape(_N, 1)
    b0r, b1r, b2r = (v.reshape(1, _D) for v in (b0, b1, b2))
    a0r, a1r = a0.reshape(1, 1), a1.reshape(1, 1)
    lbr = lin_b.reshape(1, _OUT)

    degp = _sc_deg(dst_d, zrow_16, ones_16)
    dinv, t = _tc_prescale(degp[0], degp[1], x)
    p = _sc_spmm(t, src_s, dst_s, zrow_d)
    t = _tc_layer(p, t, dinv, W0, b0r, a0r)
    p = _sc_spmm(t, src_s, dst_s, zrow_d)
    t = _tc_layer(p, t, dinv, W1, b1r, a1r)
    p = _sc_spmm(t, src_s, dst_s, zrow_d)
    return _tc_final(p, t, dinv, W2, b2r, batch2d, lin_W, lbr)


# speedup vs baseline: 24.9388x; 1.1220x over previous
import functools

import jax
import jax.numpy as jnp
from jax import lax
from jax.experimental import pallas as pl
from jax.experimental.pallas import tpu as pltpu
from jax.experimental.pallas import tpu_sc as plsc

_N, _D, _E, _G, _OUT = 10000, 128, 320000, 64, 10
_D2 = _D // 2
_NC, _NS = 2, 16
_NW = _NC * _NS
_K = 125
_EWS = _E // _NS
_STS = _EWS // _K
_EWD = _E // _NW
_STD = _EWD // _K
_NB = 4
_PF = 2
_NP = 10240
_RPT = _NP // _NS

_mesh = plsc.VectorSubcoreMesh(core_axis_name="c", subcore_axis_name="s")


def _sc_spmm_body(t_hbm, src_hbm, dst_hbm, zrow_hbm, out_hbm,
                  src_v, dst_v, rows_v, acc_sh,
                  gs0, gs1, gs2, gs3, ss0):
    c = lax.axis_index("c")
    s = lax.axis_index("s")
    gs = (gs0, gs1, gs2, gs3)
    pltpu.sync_copy(src_hbm.at[c, s], src_v)
    pltpu.sync_copy(dst_hbm.at[s], dst_v)
    pltpu.sync_copy(zrow_hbm, acc_sh.at[pl.ds(s * _RPT, _RPT)])
    plsc.subcore_barrier()

    def _gather(j, b):
        pltpu.async_copy(t_hbm.at[src_v.at[j]], rows_v.at[b], gs[b])

    def _gwait(j, b):
        pltpu.make_async_copy(t_hbm.at[src_v.at[j]], rows_v.at[b],
                              gs[b]).wait()

    def _scat(j, b):
        pltpu.async_copy(rows_v.at[b], acc_sh.at[dst_v.at[j]], ss0,
                         add=True)
        pltpu.make_async_copy(
            rows_v.at[b], acc_sh.at[dst_v.at[j]], ss0).wait()

    for j in range(_PF):
        _gather(j, j % _NB)

    def _body(i, carry):
        for b in range(_NB):
            j = _NB * i + b
            _gwait(j, b)
            _scat(j, b)
            _gather(j + _PF, (b + _PF) % _NB)
        return carry

    lax.fori_loop(0, _STS // _NB - 1, _body, 0)
    for b in range(_NB):
        j = _STS - _NB + b
        _gwait(j, b)
        _scat(j, b)
        k = j + _PF
        if k < _STS:
            _gather(k, k % _NB)

    plsc.subcore_barrier()
    pltpu.sync_copy(acc_sh.at[pl.ds(s * _RPT, _RPT)],
                    out_hbm.at[pl.ds(s * _RPT, _RPT),
                               pl.ds(c * _D2, _D2)])


def _make_sc_spmm(interpret=False):
    return functools.partial(
        pl.kernel,
        out_type=jax.ShapeDtypeStruct((_NP, _D), jnp.float32),
        mesh=_mesh,
        interpret=interpret,
        compiler_params=pltpu.CompilerParams(use_tc_tiling_on_sc=False),
        scratch_types=[
            pltpu.VMEM((_STS, _K), jnp.int32),
            pltpu.VMEM((_STS, _K), jnp.int32),
            pltpu.VMEM((_NB, _K, _D2), jnp.float32),
            pltpu.VMEM_SHARED((_NP, _D2), jnp.float32),
        ] + [pltpu.SemaphoreType.DMA] * (_NB + 1),
    )(_sc_spmm_body)


_sc_spmm = _make_sc_spmm()


def _sc_deg_body(dst_hbm, zrow_hbm, ones_hbm, out_hbm, dst_v, ones_v,
                 acc_sh):
    c = lax.axis_index("c")
    s = lax.axis_index("s")
    wid = c * _NS + s
    pltpu.sync_copy(zrow_hbm, acc_sh.at[pl.ds(s * _RPT, _RPT)])
    pltpu.sync_copy(dst_hbm.at[wid], dst_v)
    pltpu.sync_copy(ones_hbm, ones_v)
    plsc.subcore_barrier()

    def _body(j, carry):
        pltpu.sync_copy(ones_v, acc_sh.at[dst_v.at[j]], add=True)
        return carry

    lax.fori_loop(0, _STD, _body, 0)
    plsc.subcore_barrier()
    pltpu.sync_copy(acc_sh.at[pl.ds(s * _RPT, _RPT)],
                    out_hbm.at[pl.ds(s * _RPT, _RPT), pl.ds(c * 16, 16)])


def _make_sc_deg(interpret=False):
    return functools.partial(
        pl.kernel,
        out_type=jax.ShapeDtypeStruct((_NP, _D), jnp.float32),
        mesh=_mesh,
        interpret=interpret,
        compiler_params=pltpu.CompilerParams(use_tc_tiling_on_sc=False),
        scratch_types=[
            pltpu.VMEM((_STD, _K), jnp.int32),
            pltpu.VMEM((_K, 16), jnp.float32),
            pltpu.VMEM_SHARED((_NP, 16), jnp.float32),
        ],
    )(_sc_deg_body)


_sc_deg = _make_sc_deg()


_R = 1000


def _k0_body(dp_ref, x_ref, dinv_ref, t_ref):
    deg = dp_ref[:, 0:1] + dp_ref[:, 16:17] + 1.0
    dinv = lax.rsqrt(deg)
    dinv_ref[...] = dinv
    t_ref[...] = x_ref[...] * dinv


def _tc_prescale(dp, x):
    return pl.pallas_call(
        _k0_body,
        grid=(_N // _R,),
        in_specs=[
            pl.BlockSpec((_R, _D), lambda i: (i, 0)),
            pl.BlockSpec((_R, _D), lambda i: (i, 0)),
        ],
        out_specs=[
            pl.BlockSpec((_R, 1), lambda i: (i, 0)),
            pl.BlockSpec((_R, _D), lambda i: (i, 0)),
        ],
        out_shape=[
            jax.ShapeDtypeStruct((_N, 1), jnp.float32),
            jax.ShapeDtypeStruct((_N, _D), jnp.float32),
        ],
    )(dp, x)


def _klayer_body(p_ref, t_ref, dinv_ref, w_ref, b_ref, a_ref, out_ref):
    dinv = dinv_ref[...]
    sm = dinv * (p_ref[...] + t_ref[...])
    h = jnp.dot(sm, w_ref[...], preferred_element_type=jnp.float32) \
        + b_ref[...]
    a = a_ref[0, 0]
    act = jnp.where(h > 0, h, a * h)
    out_ref[...] = act * dinv


def _tc_layer(p, t, dinv, w, b, a):
    return pl.pallas_call(
        _klayer_body,
        grid=(_N // _R,),
        in_specs=[
            pl.BlockSpec((_R, _D), lambda i: (i, 0)),
            pl.BlockSpec((_R, _D), lambda i: (i, 0)),
            pl.BlockSpec((_R, 1), lambda i: (i, 0)),
            pl.BlockSpec((_D, _D), lambda i: (0, 0)),
            pl.BlockSpec((1, _D), lambda i: (0, 0)),
            pl.BlockSpec((1, 1), lambda i: (0, 0)),
        ],
        out_specs=pl.BlockSpec((_R, _D), lambda i: (i, 0)),
        out_shape=jax.ShapeDtypeStruct((_N, _D), jnp.float32),
    )(p, t, dinv, w, b, a)


def _kfinal_body(p_ref, t_ref, dinv_ref, w_ref, b_ref, batch_ref, lw_ref,
                 lb_ref, out_ref, pooled_acc, cnt_acc):
    i = pl.program_id(0)

    @pl.when(i == 0)
    def _init():
        pooled_acc[...] = jnp.zeros_like(pooled_acc)
        cnt_acc[...] = jnp.zeros_like(cnt_acc)

    dinv = dinv_ref[...]
    sm = dinv * (p_ref[...] + t_ref[...])
    h = jnp.dot(sm, w_ref[...], preferred_element_type=jnp.float32) \
        + b_ref[...]
    m = (batch_ref[...] ==
         lax.broadcasted_iota(jnp.int32, (_R, _G), 1)).astype(jnp.float32)
    dn = (((0,), (0,)), ((), ()))
    pooled_acc[...] += lax.dot_general(
        m, h, dn, preferred_element_type=jnp.float32)
    cnt_acc[...] += lax.dot_general(
        m, jnp.ones((_R, 1), jnp.float32), dn,
        preferred_element_type=jnp.float32)

    @pl.when(i == pl.num_programs(0) - 1)
    def _fin():
        pooled = pooled_acc[...] / jnp.maximum(cnt_acc[...], 1.0)
        logits = jnp.dot(pooled, lw_ref[...],
                         preferred_element_type=jnp.float32) + lb_ref[...]
        mx = jnp.max(logits, axis=1, keepdims=True)
        lse = jnp.log(jnp.sum(jnp.exp(logits - mx), axis=1,
                              keepdims=True)) + mx
        out_ref[...] = logits - lse


def _tc_final(p, t, dinv, w, b, batch2d, lw, lb):
    return pl.pallas_call(
        _kfinal_body,
        grid=(_N // _R,),
        in_specs=[
            pl.BlockSpec((_R, _D), lambda i: (i, 0)),
            pl.BlockSpec((_R, _D), lambda i: (i, 0)),
            pl.BlockSpec((_R, 1), lambda i: (i, 0)),
            pl.BlockSpec((_D, _D), lambda i: (0, 0)),
            pl.BlockSpec((1, _D), lambda i: (0, 0)),
            pl.BlockSpec((_R, 1), lambda i: (i, 0)),
            pl.BlockSpec((_D, _OUT), lambda i: (0, 0)),
            pl.BlockSpec((1, _OUT), lambda i: (0, 0)),
        ],
        out_specs=pl.BlockSpec((_G, _OUT), lambda i: (0, 0)),
        out_shape=jax.ShapeDtypeStruct((_G, _OUT), jnp.float32),
        scratch_shapes=[
            pltpu.VMEM((_G, _D), jnp.float32),
            pltpu.VMEM((_G, 1), jnp.float32),
        ],
    )(p, t, dinv, w, b, batch2d, lw, lb)


def kernel(x, edge_index, batch, W0, b0, W1, b1, W2, b2, a0, a1, lin_W,
           lin_b):
    src = edge_index[0]
    src2 = jnp.stack([2 * src, 2 * src + 1]).reshape(_NC, _NS, _STS, _K)
    dst_s = edge_index[1].reshape(_NS, _STS, _K)
    dst_d = edge_index[1].reshape(_NW, _STD, _K)
    zrow_d = jnp.zeros((_RPT, _D2), jnp.float32)
    zrow_16 = jnp.zeros((_RPT, 16), jnp.float32)
    ones_16 = jnp.ones((_K, 16), jnp.float32)
    batch2d = batch.reshape(_N, 1)
    b0r, b1r, b2r = (v.reshape(1, _D) for v in (b0, b1, b2))
    a0r, a1r = a0.reshape(1, 1), a1.reshape(1, 1)
    lbr = lin_b.reshape(1, _OUT)

    dp = _sc_deg(dst_d, zrow_16, ones_16)
    dinv, t = _tc_prescale(dp[: _N], x)
    p = _sc_spmm(t.reshape(2 * _N, _D2), src2, dst_s, zrow_d)
    t = _tc_layer(p[: _N], t, dinv, W0, b0r, a0r)
    p = _sc_spmm(t.reshape(2 * _N, _D2), src2, dst_s, zrow_d)
    t = _tc_layer(p[: _N], t, dinv, W1, b1r, a1r)
    p = _sc_spmm(t.reshape(2 * _N, _D2), src2, dst_s, zrow_d)
    return _tc_final(p[: _N], t, dinv, W2, b2r, batch2d, lin_W, lbr)


# speedup vs baseline: 25.4498x; 1.0205x over previous
import functools

import jax
import jax.numpy as jnp
from jax import lax
from jax.experimental import pallas as pl
from jax.experimental.pallas import tpu as pltpu
from jax.experimental.pallas import tpu_sc as plsc

_N, _D, _E, _G, _OUT = 10000, 128, 320000, 64, 10
_D2 = _D // 2
_NC, _NS = 2, 16
_NW = _NC * _NS
_K = 125
_EWS = _E // _NS
_STS = _EWS // _K
_EWD = _E // _NW
_STD = _EWD // _K
_NB = 4
_PF = 2
_NP = 10240
_RPT = _NP // _NS

_mesh = plsc.VectorSubcoreMesh(core_axis_name="c", subcore_axis_name="s")


def _sc_spmm_body(t_hbm, src_hbm, dst_hbm, zrow_hbm, out_hbm,
                  src_v, dst_v, rows_v, acc_sh,
                  gs0, gs1, gs2, gs3, ss0):
    c = lax.axis_index("c")
    s = lax.axis_index("s")
    gs = (gs0, gs1, gs2, gs3)
    pltpu.sync_copy(src_hbm.at[c, s], src_v)
    pltpu.sync_copy(dst_hbm.at[s], dst_v)
    pltpu.sync_copy(zrow_hbm, acc_sh.at[pl.ds(s * _RPT, _RPT)])
    plsc.subcore_barrier()

    def _gather(j, b):
        pltpu.async_copy(t_hbm.at[src_v.at[j]], rows_v.at[b], gs[b])

    def _gwait(j, b):
        pltpu.make_async_copy(t_hbm.at[src_v.at[j]], rows_v.at[b],
                              gs[b]).wait()

    def _scat(j, b):
        pltpu.async_copy(rows_v.at[b], acc_sh.at[dst_v.at[j]], ss0,
                         add=True)
        pltpu.make_async_copy(
            rows_v.at[b], acc_sh.at[dst_v.at[j]], ss0).wait()

    for j in range(_PF):
        _gather(j, j % _NB)

    def _body(i, carry):
        for b in range(_NB):
            j = _NB * i + b
            _gwait(j, b)
            _scat(j, b)
            _gather(j + _PF, (b + _PF) % _NB)
        return carry

    lax.fori_loop(0, _STS // _NB - 1, _body, 0)
    for b in range(_NB):
        j = _STS - _NB + b
        _gwait(j, b)
        _scat(j, b)
        k = j + _PF
        if k < _STS:
            _gather(k, k % _NB)

    plsc.subcore_barrier()
    pltpu.sync_copy(acc_sh.at[pl.ds(s * _RPT, _RPT)],
                    out_hbm.at[pl.ds(s * _RPT, _RPT),
                               pl.ds(c * _D2, _D2)])


def _make_sc_spmm(interpret=False):
    return functools.partial(
        pl.kernel,
        out_type=jax.ShapeDtypeStruct((_NP, _D), jnp.float32),
        mesh=_mesh,
        interpret=interpret,
        compiler_params=pltpu.CompilerParams(use_tc_tiling_on_sc=False),
        scratch_types=[
            pltpu.VMEM((_STS, _K), jnp.int32),
            pltpu.VMEM((_STS, _K), jnp.int32),
            pltpu.VMEM((_NB, _K, _D2), jnp.float32),
            pltpu.VMEM_SHARED((_NP, _D2), jnp.float32),
        ] + [pltpu.SemaphoreType.DMA] * (_NB + 1),
    )(_sc_spmm_body)


_sc_spmm = _make_sc_spmm()


def _sc_deg_body(dst_hbm, zrow_hbm, ones_hbm, out_hbm, dst_v, ones_v,
                 acc_sh):
    c = lax.axis_index("c")
    s = lax.axis_index("s")
    wid = c * _NS + s
    pltpu.sync_copy(zrow_hbm, acc_sh.at[pl.ds(s * _RPT, _RPT)])
    pltpu.sync_copy(dst_hbm.at[wid], dst_v)
    pltpu.sync_copy(ones_hbm, ones_v)
    plsc.subcore_barrier()

    def _body(j, carry):
        pltpu.sync_copy(ones_v, acc_sh.at[dst_v.at[j]], add=True)
        return carry

    lax.fori_loop(0, _STD, _body, 0)
    plsc.subcore_barrier()
    pltpu.sync_copy(acc_sh.at[pl.ds(s * _RPT, _RPT)],
                    out_hbm.at[pl.ds(s * _RPT, _RPT), pl.ds(c * 16, 16)])


def _make_sc_deg(interpret=False):
    return functools.partial(
        pl.kernel,
        out_type=jax.ShapeDtypeStruct((_NP, _D), jnp.float32),
        mesh=_mesh,
        interpret=interpret,
        compiler_params=pltpu.CompilerParams(use_tc_tiling_on_sc=False),
        scratch_types=[
            pltpu.VMEM((_STD, _K), jnp.int32),
            pltpu.VMEM((_K, 16), jnp.float32),
            pltpu.VMEM_SHARED((_NP, 16), jnp.float32),
        ],
    )(_sc_deg_body)


_sc_deg = _make_sc_deg()


_R = 2000


def _k0_body(dp_ref, x_ref, dinv_ref, t_ref):
    deg = dp_ref[:, 0:1] + dp_ref[:, 16:17] + 1.0
    dinv = lax.rsqrt(deg)
    dinv_ref[...] = dinv
    t_ref[...] = x_ref[...] * dinv


def _tc_prescale(dp, x):
    return pl.pallas_call(
        _k0_body,
        grid=(_N // _R,),
        in_specs=[
            pl.BlockSpec((_R, _D), lambda i: (i, 0)),
            pl.BlockSpec((_R, _D), lambda i: (i, 0)),
        ],
        out_specs=[
            pl.BlockSpec((_R, 1), lambda i: (i, 0)),
            pl.BlockSpec((_R, _D), lambda i: (i, 0)),
        ],
        out_shape=[
            jax.ShapeDtypeStruct((_N, 1), jnp.float32),
            jax.ShapeDtypeStruct((_N, _D), jnp.float32),
        ],
    )(dp, x)


def _klayer_body(p_ref, t_ref, dinv_ref, w_ref, b_ref, a_ref, out_ref):
    dinv = dinv_ref[...]
    sm = dinv * (p_ref[...] + t_ref[...])
    h = jnp.dot(sm, w_ref[...], preferred_element_type=jnp.float32) \
        + b_ref[...]
    a = a_ref[0, 0]
    act = jnp.where(h > 0, h, a * h)
    out_ref[...] = act * dinv


def _tc_layer(p, t, dinv, w, b, a):
    return pl.pallas_call(
        _klayer_body,
        grid=(_N // _R,),
        in_specs=[
            pl.BlockSpec((_R, _D), lambda i: (i, 0)),
            pl.BlockSpec((_R, _D), lambda i: (i, 0)),
            pl.BlockSpec((_R, 1), lambda i: (i, 0)),
            pl.BlockSpec((_D, _D), lambda i: (0, 0)),
            pl.BlockSpec((1, _D), lambda i: (0, 0)),
            pl.BlockSpec((1, 1), lambda i: (0, 0)),
        ],
        out_specs=pl.BlockSpec((_R, _D), lambda i: (i, 0)),
        out_shape=jax.ShapeDtypeStruct((_N, _D), jnp.float32),
    )(p, t, dinv, w, b, a)


def _kfinal_body(p_ref, t_ref, dinv_ref, w_ref, b_ref, batch_ref, lw_ref,
                 lb_ref, out_ref, pooled_acc, cnt_acc):
    i = pl.program_id(0)

    @pl.when(i == 0)
    def _init():
        pooled_acc[...] = jnp.zeros_like(pooled_acc)
        cnt_acc[...] = jnp.zeros_like(cnt_acc)

    dinv = dinv_ref[...]
    sm = dinv * (p_ref[...] + t_ref[...])
    h = jnp.dot(sm, w_ref[...], preferred_element_type=jnp.float32) \
        + b_ref[...]
    m = (batch_ref[...] ==
         lax.broadcasted_iota(jnp.int32, (_R, _G), 1)).astype(jnp.float32)
    dn = (((0,), (0,)), ((), ()))
    pooled_acc[...] += lax.dot_general(
        m, h, dn, preferred_element_type=jnp.float32)
    cnt_acc[...] += lax.dot_general(
        m, jnp.ones((_R, 1), jnp.float32), dn,
        preferred_element_type=jnp.float32)

    @pl.when(i == pl.num_programs(0) - 1)
    def _fin():
        pooled = pooled_acc[...] / jnp.maximum(cnt_acc[...], 1.0)
        logits = jnp.dot(pooled, lw_ref[...],
                         preferred_element_type=jnp.float32) + lb_ref[...]
        mx = jnp.max(logits, axis=1, keepdims=True)
        lse = jnp.log(jnp.sum(jnp.exp(logits - mx), axis=1,
                              keepdims=True)) + mx
        out_ref[...] = logits - lse


def _tc_final(p, t, dinv, w, b, batch2d, lw, lb):
    return pl.pallas_call(
        _kfinal_body,
        grid=(_N // _R,),
        in_specs=[
            pl.BlockSpec((_R, _D), lambda i: (i, 0)),
            pl.BlockSpec((_R, _D), lambda i: (i, 0)),
            pl.BlockSpec((_R, 1), lambda i: (i, 0)),
            pl.BlockSpec((_D, _D), lambda i: (0, 0)),
            pl.BlockSpec((1, _D), lambda i: (0, 0)),
            pl.BlockSpec((_R, 1), lambda i: (i, 0)),
            pl.BlockSpec((_D, _OUT), lambda i: (0, 0)),
            pl.BlockSpec((1, _OUT), lambda i: (0, 0)),
        ],
        out_specs=pl.BlockSpec((_G, _OUT), lambda i: (0, 0)),
        out_shape=jax.ShapeDtypeStruct((_G, _OUT), jnp.float32),
        scratch_shapes=[
            pltpu.VMEM((_G, _D), jnp.float32),
            pltpu.VMEM((_G, 1), jnp.float32),
        ],
    )(p, t, dinv, w, b, batch2d, lw, lb)


def kernel(x, edge_index, batch, W0, b0, W1, b1, W2, b2, a0, a1, lin_W,
           lin_b):
    src = edge_index[0]
    src2 = jnp.stack([2 * src, 2 * src + 1]).reshape(_NC, _NS, _STS, _K)
    dst_s = edge_index[1].reshape(_NS, _STS, _K)
    dst_d = edge_index[1].reshape(_NW, _STD, _K)
    zrow_d = jnp.zeros((_RPT, _D2), jnp.float32)
    zrow_16 = jnp.zeros((_RPT, 16), jnp.float32)
    ones_16 = jnp.ones((_K, 16), jnp.float32)
    batch2d = batch.reshape(_N, 1)
    b0r, b1r, b2r = (v.reshape(1, _D) for v in (b0, b1, b2))
    a0r, a1r = a0.reshape(1, 1), a1.reshape(1, 1)
    lbr = lin_b.reshape(1, _OUT)

    dp = _sc_deg(dst_d, zrow_16, ones_16)
    dinv, t = _tc_prescale(dp[: _N], x)
    p = _sc_spmm(t.reshape(2 * _N, _D2), src2, dst_s, zrow_d)
    t = _tc_layer(p[: _N], t, dinv, W0, b0r, a0r)
    p = _sc_spmm(t.reshape(2 * _N, _D2), src2, dst_s, zrow_d)
    t = _tc_layer(p[: _N], t, dinv, W1, b1r, a1r)
    p = _sc_spmm(t.reshape(2 * _N, _D2), src2, dst_s, zrow_d)
    return _tc_final(p[: _N], t, dinv, W2, b2r, batch2d, lin_W, lbr)


# speedup vs baseline: 31.7503x; 1.2476x over previous
import functools

import jax
import jax.numpy as jnp
from jax import lax
from jax.experimental import pallas as pl
from jax.experimental.pallas import tpu as pltpu
from jax.experimental.pallas import tpu_sc as plsc

_N, _D, _E, _G, _OUT = 10000, 128, 320000, 64, 10
_D2 = _D // 2
_NC, _NS = 2, 16
_NW = _NC * _NS
_K = 125
_EWS = _E // _NS
_STS = _EWS // _K
_EWD = _E // _NW
_STD = _EWD // _K
_NB = 4
_PF = 3
_NP = 10240
_RPT = _NP // _NS

_mesh = plsc.VectorSubcoreMesh(core_axis_name="c", subcore_axis_name="s")


def _sc_spmm_body(t_hbm, src_hbm, dst_hbm, zrow_hbm, out_hbm,
                  src_v, dst_v, rows_v, acc_sh,
                  gs0, gs1, gs2, gs3, ss0):
    c = lax.axis_index("c")
    s = lax.axis_index("s")
    gs = (gs0, gs1, gs2, gs3)
    pltpu.sync_copy(src_hbm.at[c, s], src_v)
    pltpu.sync_copy(dst_hbm.at[s], dst_v)
    pltpu.sync_copy(zrow_hbm, acc_sh.at[pl.ds(s * _RPT, _RPT)])
    plsc.subcore_barrier()

    def _gather(j, b):
        pltpu.async_copy(t_hbm.at[src_v.at[j]], rows_v.at[b], gs[b])

    def _gwait(j, b):
        pltpu.make_async_copy(t_hbm.at[src_v.at[j]], rows_v.at[b],
                              gs[b]).wait()

    def _sstart(j, b):
        pltpu.async_copy(rows_v.at[b], acc_sh.at[dst_v.at[j]], ss0,
                         add=True)

    def _swait(j, b):
        pltpu.make_async_copy(
            rows_v.at[b], acc_sh.at[dst_v.at[j]], ss0).wait()

    for j in range(_PF):
        _gather(j, j % _NB)

    def _body(i, carry):
        for b in range(_NB):
            j = _NB * i + b
            _gwait(j, b)
            _sstart(j, b)
            _gather(j + _PF, (b + _PF) % _NB)
            _swait(j, b)
        return carry

    lax.fori_loop(0, _STS // _NB - 1, _body, 0)
    for b in range(_NB):
        j = _STS - _NB + b
        _gwait(j, b)
        _sstart(j, b)
        k = j + _PF
        if k < _STS:
            _gather(k, k % _NB)
        _swait(j, b)

    plsc.subcore_barrier()
    pltpu.sync_copy(acc_sh.at[pl.ds(s * _RPT, _RPT)],
                    out_hbm.at[pl.ds(s * _RPT, _RPT),
                               pl.ds(c * _D2, _D2)])


def _make_sc_spmm(interpret=False):
    return functools.partial(
        pl.kernel,
        out_type=jax.ShapeDtypeStruct((_NP, _D), jnp.float32),
        mesh=_mesh,
        interpret=interpret,
        compiler_params=pltpu.CompilerParams(use_tc_tiling_on_sc=False),
        scratch_types=[
            pltpu.VMEM((_STS, _K), jnp.int32),
            pltpu.VMEM((_STS, _K), jnp.int32),
            pltpu.VMEM((_NB, _K, _D2), jnp.float32),
            pltpu.VMEM_SHARED((_NP, _D2), jnp.float32),
        ] + [pltpu.SemaphoreType.DMA] * (_NB + 1),
    )(_sc_spmm_body)


_sc_spmm = _make_sc_spmm()


def _sc_deg_body(dst_hbm, zrow_hbm, ones_hbm, out_hbm, dst_v, ones_v,
                 acc_sh):
    c = lax.axis_index("c")
    s = lax.axis_index("s")
    wid = c * _NS + s
    pltpu.sync_copy(zrow_hbm, acc_sh.at[pl.ds(s * _RPT, _RPT)])
    pltpu.sync_copy(dst_hbm.at[wid], dst_v)
    pltpu.sync_copy(ones_hbm, ones_v)
    plsc.subcore_barrier()

    def _body(j, carry):
        pltpu.sync_copy(ones_v, acc_sh.at[dst_v.at[j]], add=True)
        return carry

    lax.fori_loop(0, _STD, _body, 0)
    plsc.subcore_barrier()
    pltpu.sync_copy(acc_sh.at[pl.ds(s * _RPT, _RPT)],
                    out_hbm.at[pl.ds(s * _RPT, _RPT), pl.ds(c * 16, 16)])


def _make_sc_deg(interpret=False):
    return functools.partial(
        pl.kernel,
        out_type=jax.ShapeDtypeStruct((_NP, _D), jnp.float32),
        mesh=_mesh,
        interpret=interpret,
        compiler_params=pltpu.CompilerParams(use_tc_tiling_on_sc=False),
        scratch_types=[
            pltpu.VMEM((_STD, _K), jnp.int32),
            pltpu.VMEM((_K, 16), jnp.float32),
            pltpu.VMEM_SHARED((_NP, 16), jnp.float32),
        ],
    )(_sc_deg_body)


_sc_deg = _make_sc_deg()


_R = 2000


def _k0_body(dp_ref, x_ref, dinv_ref, t_ref):
    deg = dp_ref[:, 0:1] + dp_ref[:, 16:17] + 1.0
    dinv = lax.rsqrt(deg)
    dinv_ref[...] = dinv
    t_ref[...] = x_ref[...] * dinv


def _tc_prescale(dp, x):
    return pl.pallas_call(
        _k0_body,
        grid=(_N // _R,),
        in_specs=[
            pl.BlockSpec((_R, _D), lambda i: (i, 0)),
            pl.BlockSpec((_R, _D), lambda i: (i, 0)),
        ],
        out_specs=[
            pl.BlockSpec((_R, 1), lambda i: (i, 0)),
            pl.BlockSpec((_R, _D), lambda i: (i, 0)),
        ],
        out_shape=[
            jax.ShapeDtypeStruct((_N, 1), jnp.float32),
            jax.ShapeDtypeStruct((_N, _D), jnp.float32),
        ],
    )(dp, x)


def _klayer_body(p_ref, t_ref, dinv_ref, w_ref, b_ref, a_ref, out_ref):
    dinv = dinv_ref[...]
    sm = dinv * (p_ref[...] + t_ref[...])
    h = jnp.dot(sm, w_ref[...], preferred_element_type=jnp.float32) \
        + b_ref[...]
    a = a_ref[0, 0]
    act = jnp.where(h > 0, h, a * h)
    out_ref[...] = act * dinv


def _tc_layer(p, t, dinv, w, b, a):
    return pl.pallas_call(
        _klayer_body,
        grid=(_N // _R,),
        in_specs=[
            pl.BlockSpec((_R, _D), lambda i: (i, 0)),
            pl.BlockSpec((_R, _D), lambda i: (i, 0)),
            pl.BlockSpec((_R, 1), lambda i: (i, 0)),
            pl.BlockSpec((_D, _D), lambda i: (0, 0)),
            pl.BlockSpec((1, _D), lambda i: (0, 0)),
            pl.BlockSpec((1, 1), lambda i: (0, 0)),
        ],
        out_specs=pl.BlockSpec((_R, _D), lambda i: (i, 0)),
        out_shape=jax.ShapeDtypeStruct((_N, _D), jnp.float32),
    )(p, t, dinv, w, b, a)


def _kfinal_body(p_ref, t_ref, dinv_ref, w_ref, b_ref, batch_ref, lw_ref,
                 lb_ref, out_ref, pooled_acc, cnt_acc):
    i = pl.program_id(0)

    @pl.when(i == 0)
    def _init():
        pooled_acc[...] = jnp.zeros_like(pooled_acc)
        cnt_acc[...] = jnp.zeros_like(cnt_acc)

    dinv = dinv_ref[...]
    sm = dinv * (p_ref[...] + t_ref[...])
    h = jnp.dot(sm, w_ref[...], preferred_element_type=jnp.float32) \
        + b_ref[...]
    m = (batch_ref[...] ==
         lax.broadcasted_iota(jnp.int32, (_R, _G), 1)).astype(jnp.float32)
    dn = (((0,), (0,)), ((), ()))
    pooled_acc[...] += lax.dot_general(
        m, h, dn, preferred_element_type=jnp.float32)
    cnt_acc[...] += lax.dot_general(
        m, jnp.ones((_R, 1), jnp.float32), dn,
        preferred_element_type=jnp.float32)

    @pl.when(i == pl.num_programs(0) - 1)
    def _fin():
        pooled = pooled_acc[...] / jnp.maximum(cnt_acc[...], 1.0)
        logits = jnp.dot(pooled, lw_ref[...],
                         preferred_element_type=jnp.float32) + lb_ref[...]
        mx = jnp.max(logits, axis=1, keepdims=True)
        lse = jnp.log(jnp.sum(jnp.exp(logits - mx), axis=1,
                              keepdims=True)) + mx
        out_ref[...] = logits - lse


def _tc_final(p, t, dinv, w, b, batch2d, lw, lb):
    return pl.pallas_call(
        _kfinal_body,
        grid=(_N // _R,),
        in_specs=[
            pl.BlockSpec((_R, _D), lambda i: (i, 0)),
            pl.BlockSpec((_R, _D), lambda i: (i, 0)),
            pl.BlockSpec((_R, 1), lambda i: (i, 0)),
            pl.BlockSpec((_D, _D), lambda i: (0, 0)),
            pl.BlockSpec((1, _D), lambda i: (0, 0)),
            pl.BlockSpec((_R, 1), lambda i: (i, 0)),
            pl.BlockSpec((_D, _OUT), lambda i: (0, 0)),
            pl.BlockSpec((1, _OUT), lambda i: (0, 0)),
        ],
        out_specs=pl.BlockSpec((_G, _OUT), lambda i: (0, 0)),
        out_shape=jax.ShapeDtypeStruct((_G, _OUT), jnp.float32),
        scratch_shapes=[
            pltpu.VMEM((_G, _D), jnp.float32),
            pltpu.VMEM((_G, 1), jnp.float32),
        ],
    )(p, t, dinv, w, b, batch2d, lw, lb)


def kernel(x, edge_index, batch, W0, b0, W1, b1, W2, b2, a0, a1, lin_W,
           lin_b):
    src = edge_index[0]
    src2 = jnp.stack([2 * src, 2 * src + 1]).reshape(_NC, _NS, _STS, _K)
    dst_s = edge_index[1].reshape(_NS, _STS, _K)
    dst_d = edge_index[1].reshape(_NW, _STD, _K)
    zrow_d = jnp.zeros((_RPT, _D2), jnp.float32)
    zrow_16 = jnp.zeros((_RPT, 16), jnp.float32)
    ones_16 = jnp.ones((_K, 16), jnp.float32)
    batch2d = batch.reshape(_N, 1)
    b0r, b1r, b2r = (v.reshape(1, _D) for v in (b0, b1, b2))
    a0r, a1r = a0.reshape(1, 1), a1.reshape(1, 1)
    lbr = lin_b.reshape(1, _OUT)

    dp = _sc_deg(dst_d, zrow_16, ones_16)
    dinv, t = _tc_prescale(dp[: _N], x)
    p = _sc_spmm(t.reshape(2 * _N, _D2), src2, dst_s, zrow_d)
    t = _tc_layer(p[: _N], t, dinv, W0, b0r, a0r)
    p = _sc_spmm(t.reshape(2 * _N, _D2), src2, dst_s, zrow_d)
    t = _tc_layer(p[: _N], t, dinv, W1, b1r, a1r)
    p = _sc_spmm(t.reshape(2 * _N, _D2), src2, dst_s, zrow_d)
    return _tc_final(p[: _N], t, dinv, W2, b2r, batch2d, lin_W, lbr)


# speedup vs baseline: 33.0007x; 1.0394x over previous
import functools

import jax
import jax.numpy as jnp
from jax import lax
from jax.experimental import pallas as pl
from jax.experimental.pallas import tpu as pltpu
from jax.experimental.pallas import tpu_sc as plsc

_N, _D, _E, _G, _OUT = 10000, 128, 320000, 64, 10
_D2 = _D // 2
_NC, _NS = 2, 16
_NW = _NC * _NS
_K = 125
_EWS = _E // _NS
_STS = _EWS // _K
_EWD = _E // _NW
_STD = _EWD // _K
_NB = 4
_PF = 3
_NP = 10240
_RPT = _NP // _NS

_mesh = plsc.VectorSubcoreMesh(core_axis_name="c", subcore_axis_name="s")


def _sc_spmm_body(t_hbm, src_hbm, dst_hbm, zrow_hbm, out_hbm,
                  src_v, dst_v, rows_v, acc_sh,
                  gs0, gs1, gs2, gs3, ss0):
    c = lax.axis_index("c")
    s = lax.axis_index("s")
    gs = (gs0, gs1, gs2, gs3)
    pltpu.sync_copy(src_hbm.at[c, s], src_v)
    pltpu.sync_copy(dst_hbm.at[s], dst_v)
    pltpu.sync_copy(zrow_hbm, acc_sh.at[pl.ds(s * _RPT, _RPT)])
    plsc.subcore_barrier()

    def _gather(j, b):
        pltpu.async_copy(t_hbm.at[src_v.at[j]], rows_v.at[b], gs[b])

    def _gwait(j, b):
        pltpu.make_async_copy(t_hbm.at[src_v.at[j]], rows_v.at[b],
                              gs[b]).wait()

    def _sstart(j, b):
        pltpu.async_copy(rows_v.at[b], acc_sh.at[dst_v.at[j]], ss0,
                         add=True)

    def _swait(j, b):
        pltpu.make_async_copy(
            rows_v.at[b], acc_sh.at[dst_v.at[j]], ss0).wait()

    for j in range(_PF):
        _gather(j, j % _NB)

    def _body(i, carry):
        for b in range(_NB):
            j = _NB * i + b
            _gwait(j, b)
            _sstart(j, b)
            _gather(j + _PF, (b + _PF) % _NB)
            _swait(j, b)
        return carry

    lax.fori_loop(0, _STS // _NB - 1, _body, 0)
    for b in range(_NB):
        j = _STS - _NB + b
        _gwait(j, b)
        _sstart(j, b)
        k = j + _PF
        if k < _STS:
            _gather(k, k % _NB)
        _swait(j, b)

    plsc.subcore_barrier()
    pltpu.sync_copy(acc_sh.at[pl.ds(s * _RPT, _RPT)],
                    out_hbm.at[pl.ds(s * _RPT, _RPT),
                               pl.ds(c * _D2, _D2)])


def _make_sc_spmm(interpret=False):
    return functools.partial(
        pl.kernel,
        out_type=jax.ShapeDtypeStruct((_NP, _D), jnp.float32),
        mesh=_mesh,
        interpret=interpret,
        compiler_params=pltpu.CompilerParams(use_tc_tiling_on_sc=False),
        scratch_types=[
            pltpu.VMEM((_STS, _K), jnp.int32),
            pltpu.VMEM((_STS, _K), jnp.int32),
            pltpu.VMEM((_NB, _K, _D2), jnp.float32),
            pltpu.VMEM_SHARED((_NP, _D2), jnp.float32),
        ] + [pltpu.SemaphoreType.DMA] * (_NB + 1),
    )(_sc_spmm_body)


_sc_spmm = _make_sc_spmm()


def _sc_deg_body(dst_hbm, zrow_hbm, ones_hbm, out_hbm, dst_v, ones_v,
                 acc_sh):
    c = lax.axis_index("c")
    s = lax.axis_index("s")
    wid = c * _NS + s
    pltpu.sync_copy(zrow_hbm, acc_sh.at[pl.ds(s * _RPT, _RPT)])
    pltpu.sync_copy(dst_hbm.at[wid], dst_v)
    pltpu.sync_copy(ones_hbm, ones_v)
    plsc.subcore_barrier()

    def _body(j, carry):
        pltpu.sync_copy(ones_v, acc_sh.at[dst_v.at[j]], add=True)
        return carry

    lax.fori_loop(0, _STD, _body, 0)
    plsc.subcore_barrier()
    pltpu.sync_copy(acc_sh.at[pl.ds(s * _RPT, _RPT)],
                    out_hbm.at[pl.ds(s * _RPT, _RPT), pl.ds(c * 16, 16)])


def _make_sc_deg(interpret=False):
    return functools.partial(
        pl.kernel,
        out_type=jax.ShapeDtypeStruct((_NP, _D), jnp.float32),
        mesh=_mesh,
        interpret=interpret,
        compiler_params=pltpu.CompilerParams(use_tc_tiling_on_sc=False),
        scratch_types=[
            pltpu.VMEM((_STD, _K), jnp.int32),
            pltpu.VMEM((_K, 16), jnp.float32),
            pltpu.VMEM_SHARED((_NP, 16), jnp.float32),
        ],
    )(_sc_deg_body)


_sc_deg = _make_sc_deg()


_R = 2000


def _k0_body(dp_ref, x_ref, dinv_ref, t_ref):
    deg = dp_ref[:, 0:1] + dp_ref[:, 16:17] + 1.0
    dinv = lax.rsqrt(deg)
    dinv_ref[...] = dinv
    t_ref[...] = x_ref[...] * dinv


def _tc_prescale(dp, x):
    return pl.pallas_call(
        _k0_body,
        grid=(_N // _R,),
        in_specs=[
            pl.BlockSpec((_R, _D), lambda i: (i, 0)),
            pl.BlockSpec((_R, _D), lambda i: (i, 0)),
        ],
        out_specs=[
            pl.BlockSpec((_R, 1), lambda i: (i, 0)),
            pl.BlockSpec((_R, _D), lambda i: (i, 0)),
        ],
        out_shape=[
            jax.ShapeDtypeStruct((_N, 1), jnp.float32),
            jax.ShapeDtypeStruct((_N, _D), jnp.float32),
        ],
    )(dp, x)


def _klayer_body(p_ref, t_ref, dinv_ref, w_ref, b_ref, a_ref, out_ref):
    dinv = dinv_ref[...]
    sm = dinv * (p_ref[...] + t_ref[...])
    h = jnp.dot(sm, w_ref[...], preferred_element_type=jnp.float32) \
        + b_ref[...]
    a = a_ref[0, 0]
    act = jnp.where(h > 0, h, a * h)
    out_ref[...] = act * dinv


def _tc_layer(p, t, dinv, w, b, a):
    return pl.pallas_call(
        _klayer_body,
        grid=(_N // _R,),
        in_specs=[
            pl.BlockSpec((_R, _D), lambda i: (i, 0)),
            pl.BlockSpec((_R, _D), lambda i: (i, 0)),
            pl.BlockSpec((_R, 1), lambda i: (i, 0)),
            pl.BlockSpec((_D, _D), lambda i: (0, 0)),
            pl.BlockSpec((1, _D), lambda i: (0, 0)),
            pl.BlockSpec((1, 1), lambda i: (0, 0)),
        ],
        out_specs=pl.BlockSpec((_R, _D), lambda i: (i, 0)),
        out_shape=jax.ShapeDtypeStruct((_N, _D), jnp.float32),
    )(p, t, dinv, w, b, a)


def _kfinal_body(p_ref, t_ref, dinv_ref, w_ref, b_ref, batch_ref, lw_ref,
                 lb_ref, out_ref, pooled_acc, cnt_acc):
    i = pl.program_id(0)

    @pl.when(i == 0)
    def _init():
        pooled_acc[...] = jnp.zeros_like(pooled_acc)
        cnt_acc[...] = jnp.zeros_like(cnt_acc)

    dinv = dinv_ref[...]
    sm = dinv * (p_ref[...] + t_ref[...])
    h = jnp.dot(sm, w_ref[...], preferred_element_type=jnp.float32) \
        + b_ref[...]
    m = (batch_ref[...] ==
         lax.broadcasted_iota(jnp.int32, (_R, _G), 1)).astype(jnp.float32)
    dn = (((0,), (0,)), ((), ()))
    pooled_acc[...] += lax.dot_general(
        m, h, dn, preferred_element_type=jnp.float32)
    cnt_acc[...] += lax.dot_general(
        m, jnp.ones((_R, 1), jnp.float32), dn,
        preferred_element_type=jnp.float32)

    @pl.when(i == pl.num_programs(0) - 1)
    def _fin():
        pooled = pooled_acc[...] / jnp.maximum(cnt_acc[...], 1.0)
        logits = jnp.dot(pooled, lw_ref[...],
                         preferred_element_type=jnp.float32) + lb_ref[...]
        mx = jnp.max(logits, axis=1, keepdims=True)
        lse = jnp.log(jnp.sum(jnp.exp(logits - mx), axis=1,
                              keepdims=True)) + mx
        out_ref[...] = logits - lse


def _tc_final(p, t, dinv, w, b, batch2d, lw, lb):
    return pl.pallas_call(
        _kfinal_body,
        grid=(_N // _R,),
        in_specs=[
            pl.BlockSpec((_R, _D), lambda i: (i, 0)),
            pl.BlockSpec((_R, _D), lambda i: (i, 0)),
            pl.BlockSpec((_R, 1), lambda i: (i, 0)),
            pl.BlockSpec((_D, _D), lambda i: (0, 0)),
            pl.BlockSpec((1, _D), lambda i: (0, 0)),
            pl.BlockSpec((_R, 1), lambda i: (i, 0)),
            pl.BlockSpec((_D, _OUT), lambda i: (0, 0)),
            pl.BlockSpec((1, _OUT), lambda i: (0, 0)),
        ],
        out_specs=pl.BlockSpec((_G, _OUT), lambda i: (0, 0)),
        out_shape=jax.ShapeDtypeStruct((_G, _OUT), jnp.float32),
        scratch_shapes=[
            pltpu.VMEM((_G, _D), jnp.float32),
            pltpu.VMEM((_G, 1), jnp.float32),
        ],
    )(p, t, dinv, w, b, batch2d, lw, lb)


def kernel(x, edge_index, batch, W0, b0, W1, b1, W2, b2, a0, a1, lin_W,
           lin_b):
    src = edge_index[0]
    src2 = jnp.stack([2 * src, 2 * src + 1]).reshape(_NC, _NS, _STS, _K)
    dst_s = edge_index[1].reshape(_NS, _STS, _K)
    dst_d = edge_index[1].reshape(_NW, _STD, _K)
    zrow_d = jnp.zeros((_RPT, _D2), jnp.float32)
    zrow_16 = jnp.zeros((_RPT, 16), jnp.float32)
    ones_16 = jnp.ones((_K, 16), jnp.float32)
    batch2d = batch.reshape(_N, 1)
    b0r, b1r, b2r = (v.reshape(1, _D) for v in (b0, b1, b2))
    a0r, a1r = a0.reshape(1, 1), a1.reshape(1, 1)
    lbr = lin_b.reshape(1, _OUT)

    dp = _sc_deg(dst_d, zrow_16, ones_16)
    dinv, t = _tc_prescale(dp, x)
    p = _sc_spmm(t.reshape(2 * _N, _D2), src2, dst_s, zrow_d)
    t = _tc_layer(p, t, dinv, W0, b0r, a0r)
    p = _sc_spmm(t.reshape(2 * _N, _D2), src2, dst_s, zrow_d)
    t = _tc_layer(p, t, dinv, W1, b1r, a1r)
    p = _sc_spmm(t.reshape(2 * _N, _D2), src2, dst_s, zrow_d)
    return _tc_final(p, t, dinv, W2, b2r, batch2d, lin_W, lbr)
